# Initial kernel scaffold; baseline (speedup 1.0000x reference)
import jax
import jax.numpy as jnp
from jax.experimental import pallas as pl


def kernel(query, keys, values, k):
    raise NotImplementedError("write your pallas kernel here")


# speedup vs baseline: 1.2197x; 1.2197x over previous
import functools

import jax
import jax.numpy as jnp
from jax import lax
from jax.experimental import pallas as pl
from jax.experimental.pallas import tpu as pltpu
from jax.experimental.pallas import tpu_sc as plsc

D_MODEL = 64
N = 1000000
K = 16
BLK = 8000
G = N // BLK
SUB = BLK // D_MODEL

_NEG_INF = float("-inf")
_BIG_I32 = 2**31 - 1


def _sims_topk_body(q_ref, k_ref, sims_out, idx_out, s_scr, bmax_s, bidx_s):
    i = pl.program_id(0)

    q = q_ref[...]
    qn = q / jnp.maximum(jnp.sqrt(jnp.sum(q * q)), 1e-12)

    kblk = k_ref[...]
    dots = jnp.sum(kblk * qn[None], axis=2)
    nrm2 = jnp.sum(kblk * kblk, axis=2)
    s3 = dots / jnp.maximum(jnp.sqrt(nrm2), 1e-12)
    s_scr[i] = s3

    lin = (lax.broadcasted_iota(jnp.int32, (SUB, D_MODEL), 0) * D_MODEL
           + lax.broadcasted_iota(jnp.int32, (SUB, D_MODEL), 1))
    m = jnp.max(s3)
    lidx = jnp.min(jnp.where(s3 == m, lin, _BIG_I32))
    bmax_s[i] = m
    bidx_s[i] = i * BLK + lidx

    @pl.when(i == G - 1)
    def _final():
        for j in range(K):
            def scan_blocks(g, carry):
                bv, bg = carry
                v = bmax_s[g]
                take = v > bv
                return (jnp.where(take, v, bv),
                        jnp.where(take, g, bg))
            bestv, bestg = lax.fori_loop(
                0, G, scan_blocks, (jnp.float32(_NEG_INF), jnp.int32(0)))
            besti = bidx_s[bestg]
            sims_out[j] = bestv
            idx_out[j] = besti

            row = s_scr[bestg]
            off = besti - bestg * BLK
            row = jnp.where(lin == off, _NEG_INF, row)
            s_scr[bestg] = row
            nm = jnp.max(row)
            nidx = jnp.min(jnp.where(row == nm, lin, _BIG_I32))
            bmax_s[bestg] = nm
            bidx_s[bestg] = bestg * BLK + nidx


def _sims_topk(query, keys):
    return pl.pallas_call(
        _sims_topk_body,
        grid=(G,),
        in_specs=[
            pl.BlockSpec((1, D_MODEL), lambda i: (0, 0)),
            pl.BlockSpec((SUB, D_MODEL, D_MODEL), lambda i: (i, 0, 0)),
        ],
        out_specs=[
            pl.BlockSpec(memory_space=pltpu.SMEM),
            pl.BlockSpec(memory_space=pltpu.SMEM),
        ],
        out_shape=[
            jax.ShapeDtypeStruct((K,), jnp.float32),
            jax.ShapeDtypeStruct((K,), jnp.int32),
        ],
        scratch_shapes=[
            pltpu.VMEM((G, SUB, D_MODEL), jnp.float32),
            pltpu.SMEM((G,), jnp.float32),
            pltpu.SMEM((G,), jnp.int32),
        ],
    )(query.reshape(1, D_MODEL), keys.reshape(G * SUB, D_MODEL, D_MODEL))


def _sc_gather(values, idx):
    mesh = plsc.VectorSubcoreMesh(core_axis_name="c", subcore_axis_name="s")

    @functools.partial(
        pl.kernel,
        mesh=mesh,
        out_type=jax.ShapeDtypeStruct((K, D_MODEL), jnp.float32),
        scratch_types=[
            pltpu.VMEM((K,), jnp.int32),
            pltpu.VMEM((K, D_MODEL), jnp.float32),
            pltpu.SemaphoreType.DMA,
        ],
        compiler_params=pltpu.CompilerParams(use_tc_tiling_on_sc=False),
    )
    def gather_k(values_hbm, idx_hbm, out_hbm, idx_v, rows_v, sem):
        c = lax.axis_index("c")
        s = lax.axis_index("s")
        wid = s * 2 + c

        @pl.when(wid == 0)
        def _():
            pltpu.sync_copy(idx_hbm, idx_v)
            pltpu.async_copy(values_hbm.at[idx_v], rows_v, sem).wait()
            pltpu.sync_copy(rows_v, out_hbm)

    return gather_k(values, idx)


def kernel(query, keys, values, k):
    topk_sims, topk_idx = _sims_topk(query, keys)
    valid = jnp.arange(K) < k
    topk_sims = jnp.where(valid, topk_sims, topk_sims[0])
    topk_idx = jnp.where(valid, topk_idx, topk_idx[0])
    topk_values = _sc_gather(values, topk_idx)
    return topk_values, topk_sims


# speedup vs baseline: 1.4534x; 1.1916x over previous
import functools

import jax
import jax.numpy as jnp
from jax import lax
from jax.experimental import pallas as pl
from jax.experimental.pallas import tpu as pltpu
from jax.experimental.pallas import tpu_sc as plsc

D_MODEL = 64
N = 1000000
K = 16
BLK = 8000
G = N // BLK
SUB = BLK // D_MODEL

_NEG_INF = float("-inf")
_BIG_I32 = 2**31 - 1


def _sims_topk_body(q_ref, k_ref, sims_out, idx_out, s_scr, bmax_s, bidx_s):
    i = pl.program_id(0)

    q = q_ref[...]
    qn = q / jnp.maximum(jnp.sqrt(jnp.sum(q * q)), 1e-12)

    kblk = k_ref[...]
    dn = (((1,), (1,)), ((), ()))
    dots = lax.dot_general(qn, kblk, dn,
                           preferred_element_type=jnp.float32)
    ones = jnp.ones((1, D_MODEL), jnp.float32)
    nrm2 = lax.dot_general(ones, kblk * kblk, dn,
                           preferred_element_type=jnp.float32)
    s3 = dots / jnp.maximum(jnp.sqrt(nrm2), 1e-12)
    s_scr[pl.ds(i, 1)] = s3

    lin = lax.broadcasted_iota(jnp.int32, (1, BLK), 1)
    m = jnp.max(s3)
    lidx = jnp.min(jnp.where(s3 == m, lin, _BIG_I32))
    bmax_s[i] = m
    bidx_s[i] = i * BLK + lidx

    @pl.when(i == G - 1)
    def _final():
        for j in range(K):
            def scan_blocks(g, carry):
                bv, bg = carry
                v = bmax_s[g]
                take = v > bv
                return (jnp.where(take, v, bv),
                        jnp.where(take, g, bg))
            bestv, bestg = lax.fori_loop(
                0, G, scan_blocks, (jnp.float32(_NEG_INF), jnp.int32(0)))
            besti = bidx_s[bestg]
            sims_out[j] = bestv
            idx_out[j] = besti

            row = s_scr[pl.ds(bestg, 1)]
            off = besti - bestg * BLK
            row = jnp.where(lin == off, _NEG_INF, row)
            s_scr[pl.ds(bestg, 1)] = row
            nm = jnp.max(row)
            nidx = jnp.min(jnp.where(row == nm, lin, _BIG_I32))
            bmax_s[bestg] = nm
            bidx_s[bestg] = bestg * BLK + nidx


def _sims_topk(query, keys):
    return pl.pallas_call(
        _sims_topk_body,
        grid=(G,),
        in_specs=[
            pl.BlockSpec((1, D_MODEL), lambda i: (0, 0)),
            pl.BlockSpec((BLK, D_MODEL), lambda i: (i, 0)),
        ],
        out_specs=[
            pl.BlockSpec(memory_space=pltpu.SMEM),
            pl.BlockSpec(memory_space=pltpu.SMEM),
        ],
        out_shape=[
            jax.ShapeDtypeStruct((K,), jnp.float32),
            jax.ShapeDtypeStruct((K,), jnp.int32),
        ],
        scratch_shapes=[
            pltpu.VMEM((G, BLK), jnp.float32),
            pltpu.SMEM((G,), jnp.float32),
            pltpu.SMEM((G,), jnp.int32),
        ],
    )(query.reshape(1, D_MODEL), keys)


def _sc_gather(values, idx):
    mesh = plsc.VectorSubcoreMesh(core_axis_name="c", subcore_axis_name="s")

    @functools.partial(
        pl.kernel,
        mesh=mesh,
        out_type=jax.ShapeDtypeStruct((K, D_MODEL), jnp.float32),
        scratch_types=[
            pltpu.VMEM((K,), jnp.int32),
            pltpu.VMEM((K, D_MODEL), jnp.float32),
            pltpu.SemaphoreType.DMA,
        ],
        compiler_params=pltpu.CompilerParams(use_tc_tiling_on_sc=False),
    )
    def gather_k(values_hbm, idx_hbm, out_hbm, idx_v, rows_v, sem):
        c = lax.axis_index("c")
        s = lax.axis_index("s")
        wid = s * 2 + c

        @pl.when(wid == 0)
        def _():
            pltpu.sync_copy(idx_hbm, idx_v)
            pltpu.async_copy(values_hbm.at[idx_v], rows_v, sem).wait()
            pltpu.sync_copy(rows_v, out_hbm)

    return gather_k(values, idx)


def kernel(query, keys, values, k):
    topk_sims, topk_idx = _sims_topk(query, keys)
    valid = jnp.arange(K) < k
    topk_sims = jnp.where(valid, topk_sims, topk_sims[0])
    topk_idx = jnp.where(valid, topk_idx, topk_idx[0])
    topk_values = _sc_gather(values, topk_idx)
    return topk_values, topk_sims
